# Initial kernel scaffold; baseline (speedup 1.0000x reference)
import jax
import jax.numpy as jnp
from jax.experimental import pallas as pl


def kernel(input_text, text_len, emb_table, W1, b1, W2, b2):
    raise NotImplementedError("write your pallas kernel here")


# speedup vs baseline: 1.5452x; 1.5452x over previous
import functools

import jax
import jax.numpy as jnp
from jax import lax
from jax.experimental import pallas as pl
from jax.experimental.pallas import tpu as pltpu
from jax.experimental.pallas import tpu_sc as plsc

VOCAB = 100000
EMB = 300
EPAD = 304
HID = 100
NCLS = 1000
B = 4096
L = 200
HALF = L // 2
NVR = EPAD // 16

NC, NS = 2, 16
NW = NC * NS
BPW = B // NW


def _make_pool_kernel():
    mesh = plsc.VectorSubcoreMesh(core_axis_name="c", subcore_axis_name="s")

    @functools.partial(
        pl.kernel,
        mesh=mesh,
        out_type=jax.ShapeDtypeStruct((B, EPAD), jnp.float32),
        compiler_params=pltpu.CompilerParams(use_tc_tiling_on_sc=False),
        scratch_types=[
            pltpu.VMEM((BPW, 2, HALF), jnp.int32),
            pltpu.VMEM((HALF, EPAD), jnp.float32),
            pltpu.VMEM((HALF, EPAD), jnp.float32),
            pltpu.VMEM((EPAD,), jnp.float32),
            pltpu.SemaphoreType.DMA,
            pltpu.SemaphoreType.DMA,
        ],
    )
    def pool(idx_hbm, tbl_hbm, out_hbm, idx_v, gbuf0, gbuf1, orow, sem0, sem1):
        wid = lax.axis_index("s") * NC + lax.axis_index("c")
        base = wid * BPW
        pltpu.sync_copy(idx_hbm.at[pl.ds(base, BPW)], idx_v)

        def sum_rows(gbuf, acc):
            def row_body(r, a):
                return tuple(
                    a[j] + gbuf[r, pl.ds(16 * j, 16)] for j in range(NVR)
                )
            return lax.fori_loop(0, HALF, row_body, acc)

        def batch_body(b, carry):
            cp0 = pltpu.async_copy(tbl_hbm.at[idx_v.at[b, 0]], gbuf0, sem0)
            cp1 = pltpu.async_copy(tbl_hbm.at[idx_v.at[b, 1]], gbuf1, sem1)
            acc = tuple(jnp.zeros((16,), jnp.float32) for _ in range(NVR))
            cp0.wait()
            acc = sum_rows(gbuf0, acc)
            cp1.wait()
            acc = sum_rows(gbuf1, acc)
            for j in range(NVR):
                orow[pl.ds(16 * j, 16)] = acc[j]
            pltpu.sync_copy(orow, out_hbm.at[base + b])
            return carry

        lax.fori_loop(0, BPW, batch_body, 0)

    return pool


_pool = _make_pool_kernel()

BLK = 512


def _mlp_body(enc_ref, tl_ref, w1_ref, b1_ref, w2_ref, b2_ref, out_ref):
    x = enc_ref[...] / tl_ref[...]
    h = jnp.dot(x, w1_ref[...], preferred_element_type=jnp.float32) + b1_ref[...]
    h = jnp.where(h > 0, h, jnp.exp(h) - 1.0)
    out_ref[...] = (
        jnp.dot(h, w2_ref[...], preferred_element_type=jnp.float32) + b2_ref[...]
    )


def kernel(input_text, text_len, emb_table, W1, b1, W2, b2):
    idx3 = input_text.astype(jnp.int32).reshape(B, 2, HALF)
    tbl = jnp.pad(emb_table, ((0, 0), (0, EPAD - EMB)))
    tbl = tbl.at[0].set(0.0)
    w1t = jnp.pad(W1, ((0, 0), (0, EPAD - EMB))).T
    w2t = W2.T
    b1r = b1.reshape(1, HID)
    b2r = b2.reshape(1, NCLS)
    tl2 = text_len.reshape(B, 1)

    encoded = _pool(idx3, tbl)

    logits = pl.pallas_call(
        _mlp_body,
        grid=(B // BLK,),
        in_specs=[
            pl.BlockSpec((BLK, EPAD), lambda i: (i, 0)),
            pl.BlockSpec((BLK, 1), lambda i: (i, 0)),
            pl.BlockSpec((EPAD, HID), lambda i: (0, 0)),
            pl.BlockSpec((1, HID), lambda i: (0, 0)),
            pl.BlockSpec((HID, NCLS), lambda i: (0, 0)),
            pl.BlockSpec((1, NCLS), lambda i: (0, 0)),
        ],
        out_specs=pl.BlockSpec((BLK, NCLS), lambda i: (i, 0)),
        out_shape=jax.ShapeDtypeStruct((B, NCLS), jnp.float32),
    )(encoded, tl2, w1t, b1r, w2t, b2r)
    return logits


# speedup vs baseline: 2.0060x; 1.2982x over previous
import functools

import jax
import jax.numpy as jnp
from jax import lax
from jax.experimental import pallas as pl
from jax.experimental.pallas import tpu as pltpu
from jax.experimental.pallas import tpu_sc as plsc

VOCAB = 100000
EMB = 300
EPAD = 304
HID = 100
NCLS = 1000
B = 4096
L = 200
HALF = L // 2
NVR = EPAD // 16

NC, NS = 2, 16
NW = NC * NS
BPW = B // NW


def _make_pool_kernel():
    mesh = plsc.VectorSubcoreMesh(core_axis_name="c", subcore_axis_name="s")

    @functools.partial(
        pl.kernel,
        mesh=mesh,
        out_type=jax.ShapeDtypeStruct((B, EPAD), jnp.float32),
        compiler_params=pltpu.CompilerParams(use_tc_tiling_on_sc=False),
        scratch_types=[
            pltpu.VMEM((BPW, 2, HALF), jnp.int32),
            pltpu.VMEM((HALF, EPAD), jnp.float32),
            pltpu.VMEM((HALF, EPAD), jnp.float32),
            pltpu.VMEM((EPAD,), jnp.float32),
            pltpu.SemaphoreType.DMA,
            pltpu.SemaphoreType.DMA,
        ],
    )
    def pool(idx_hbm, tbl_hbm, out_hbm, idx_v, gbuf0, gbuf1, orow, sem0, sem1):
        wid = lax.axis_index("s") * NC + lax.axis_index("c")
        base = wid * BPW
        pltpu.sync_copy(idx_hbm.at[pl.ds(base, BPW)], idx_v)

        def sum_rows(gbuf, acc):
            def row_body(r, a):
                return tuple(
                    a[j] + gbuf[r, pl.ds(16 * j, 16)] for j in range(NVR)
                )
            return lax.fori_loop(0, HALF, row_body, acc)

        def batch_body(b, carry):
            cp0 = pltpu.async_copy(tbl_hbm.at[idx_v.at[b, 0]], gbuf0, sem0)
            cp1 = pltpu.async_copy(tbl_hbm.at[idx_v.at[b, 1]], gbuf1, sem1)
            acc = tuple(jnp.zeros((16,), jnp.float32) for _ in range(NVR))
            cp0.wait()
            acc = sum_rows(gbuf0, acc)
            cp1.wait()
            acc = sum_rows(gbuf1, acc)
            for j in range(NVR):
                orow[pl.ds(16 * j, 16)] = acc[j]
            pltpu.sync_copy(orow, out_hbm.at[base + b])
            return carry

        lax.fori_loop(0, BPW, batch_body, 0)

    return pool


_pool = _make_pool_kernel()

PADBLK = 2000


def _pad_body(src_ref, dst_ref):
    dst_ref[:, :EMB] = src_ref[...]
    dst_ref[:, EMB:] = jnp.zeros((PADBLK, EPAD - EMB), jnp.float32)


def _pad_table(tbl):
    return pl.pallas_call(
        _pad_body,
        grid=(VOCAB // PADBLK,),
        in_specs=[pl.BlockSpec((PADBLK, EMB), lambda i: (i, 0))],
        out_specs=pl.BlockSpec((PADBLK, EPAD), lambda i: (i, 0)),
        out_shape=jax.ShapeDtypeStruct((VOCAB, EPAD), jnp.float32),
    )(tbl)


BLK = 512


def _mlp_body(enc_ref, tl_ref, w1_ref, b1_ref, w2_ref, b2_ref, out_ref):
    x = enc_ref[...] / tl_ref[...]
    h = jnp.dot(x, w1_ref[...], preferred_element_type=jnp.float32) + b1_ref[...]
    h = jnp.where(h > 0, h, jnp.exp(h) - 1.0)
    out_ref[...] = (
        jnp.dot(h, w2_ref[...], preferred_element_type=jnp.float32) + b2_ref[...]
    )


def kernel(input_text, text_len, emb_table, W1, b1, W2, b2):
    idx3 = input_text.astype(jnp.int32).reshape(B, 2, HALF)
    w1t = jnp.pad(W1, ((0, 0), (0, EPAD - EMB))).T
    w2t = W2.T
    b1r = b1.reshape(1, HID)
    b2r = b2.reshape(1, NCLS)
    tl2 = text_len.reshape(B, 1)

    tbl = _pad_table(emb_table)
    encoded = _pool(idx3, tbl)

    logits = pl.pallas_call(
        _mlp_body,
        grid=(B // BLK,),
        in_specs=[
            pl.BlockSpec((BLK, EPAD), lambda i: (i, 0)),
            pl.BlockSpec((BLK, 1), lambda i: (i, 0)),
            pl.BlockSpec((EPAD, HID), lambda i: (0, 0)),
            pl.BlockSpec((1, HID), lambda i: (0, 0)),
            pl.BlockSpec((HID, NCLS), lambda i: (0, 0)),
            pl.BlockSpec((1, NCLS), lambda i: (0, 0)),
        ],
        out_specs=pl.BlockSpec((BLK, NCLS), lambda i: (i, 0)),
        out_shape=jax.ShapeDtypeStruct((B, NCLS), jnp.float32),
    )(encoded, tl2, w1t, b1r, w2t, b2r)
    return logits
